# baseline (speedup 1.0000x reference)
import functools

import jax
import jax.numpy as jnp
from jax import lax
from jax.experimental import pallas as pl
from jax.experimental.pallas import tpu as pltpu, tpu_sc as plsc

E = 8
D = 768
NSL = D // 16

NC = 2
NS = 16
NW = NC * NS

RC = 64
RG = 4

BLOCK = 2048


def _tc_block_kernel(x_ref, wr_ref, w_ref, b_ref, o_ref):
    x = x_ref[...]
    logits = jnp.dot(x, wr_ref[...], preferred_element_type=jnp.float32)
    values = jnp.argmax(logits, axis=-1)
    eids = jax.lax.broadcasted_iota(jnp.int32, (x.shape[0], E), 1)
    onehot = (values[:, None] == eids).astype(jnp.float32)
    w_eff = jnp.dot(onehot, w_ref[...], preferred_element_type=jnp.float32)
    b_eff = jnp.dot(onehot, b_ref[...], preferred_element_type=jnp.float32)
    o_ref[...] = x * w_eff + b_eff


def _tc_kernel(inpt, W_router, w, b):
    n, d = inpt.shape
    return pl.pallas_call(
        _tc_block_kernel,
        grid=(n // BLOCK,),
        in_specs=[
            pl.BlockSpec((BLOCK, d), lambda i: (i, 0)),
            pl.BlockSpec((d, E), lambda i: (0, 0)),
            pl.BlockSpec((E, d), lambda i: (0, 0)),
            pl.BlockSpec((E, d), lambda i: (0, 0)),
        ],
        out_specs=pl.BlockSpec((BLOCK, d), lambda i: (i, 0)),
        out_shape=jax.ShapeDtypeStruct((n, d), jnp.float32),
    )(inpt, W_router, w, b)


def _sc_body(x_hbm, wrT_hbm, w_hbm, b_hbm, out_hbm,
             xbuf, obuf, wrT_v, w_v, b_v):
    n = x_hbm.shape[0]
    rpw = n // NW
    nchunks = rpw // RC
    wid = lax.axis_index("s") * NC + lax.axis_index("c")

    pltpu.sync_copy(wrT_hbm, wrT_v)
    pltpu.sync_copy(w_hbm, w_v)
    pltpu.sync_copy(b_hbm, b_v)

    def chunk_body(c, _):
        base = wid * rpw + c * RC
        pltpu.sync_copy(x_hbm.at[pl.ds(base, RC)], xbuf)

        def group_body(g, _):
            r0 = g * RG
            acc = [[jnp.zeros((16,), jnp.float32) for _ in range(E)]
                   for _ in range(RG)]
            for j in range(NSL):
                sl = pl.ds(j * 16, 16)
                wr = [wrT_v[e, sl] for e in range(E)]
                for q in range(RG):
                    u = lax.bitcast_convert_type(xbuf[r0 + q, sl], jnp.uint32)
                    u = (u + jnp.uint32(0x7FFF) + ((u >> 16) & jnp.uint32(1)))
                    u = u & jnp.uint32(0xFFFF0000)
                    xv = lax.bitcast_convert_type(u, jnp.float32)
                    for e in range(E):
                        acc[q][e] = acc[q][e] + xv * wr[e]
            for q in range(RG):
                s = [jnp.sum(acc[q][e]) for e in range(E)]
                best = s[0]
                beste = jnp.int32(0)
                for e in range(1, E):
                    upd = s[e] > best
                    best = jnp.where(upd, s[e], best)
                    beste = jnp.where(upd, jnp.int32(e), beste)
                for j in range(NSL):
                    sl = pl.ds(j * 16, 16)
                    obuf[r0 + q, sl] = (xbuf[r0 + q, sl] * w_v[beste, sl]
                                        + b_v[beste, sl])
            return 0

        lax.fori_loop(0, RC // RG, group_body, 0)
        pltpu.sync_copy(obuf, out_hbm.at[pl.ds(base, RC)])
        return 0

    lax.fori_loop(0, nchunks, chunk_body, 0)


def _sc_kernel(inpt, W_router, w, b):
    n, d = inpt.shape
    wrT = W_router.T.reshape(E, d)
    u = lax.bitcast_convert_type(wrT, jnp.uint32)
    u = (u + jnp.uint32(0x7FFF) + ((u >> 16) & jnp.uint32(1))) & jnp.uint32(0xFFFF0000)
    wrT = lax.bitcast_convert_type(u, jnp.float32)
    mesh = plsc.VectorSubcoreMesh(core_axis_name="c", subcore_axis_name="s",
                                  num_cores=NC, num_subcores=NS)
    run = pl.kernel(
        _sc_body,
        out_type=jax.ShapeDtypeStruct((n, d), jnp.float32),
        mesh=mesh,
        scratch_types=[
            pltpu.VMEM((RC, d), jnp.float32),
            pltpu.VMEM((RC, d), jnp.float32),
            pltpu.VMEM((E, d), jnp.float32),
            pltpu.VMEM((E, d), jnp.float32),
            pltpu.VMEM((E, d), jnp.float32),
        ],
        compiler_params=pltpu.CompilerParams(needs_layout_passes=False),
    )
    return run(inpt, wrT, w, b)


def kernel(inpt, W_router, w, b):
    return _sc_kernel(inpt, W_router, w, b)


# speedup vs baseline: 11.8539x; 11.8539x over previous
import functools

import jax
import jax.numpy as jnp
from jax import lax
from jax.experimental import pallas as pl
from jax.experimental.pallas import tpu as pltpu, tpu_sc as plsc

E = 8
D = 768
NSL = D // 16

NC = 2
NS = 16
NW = NC * NS

RC = 64
RG = 4

BLOCK = 1024


def _tc_block_kernel(x_ref, wr_ref, w_ref, b_ref, o_ref):
    x = x_ref[...]
    logits = jnp.dot(x, wr_ref[...], preferred_element_type=jnp.float32)
    values = jnp.argmax(logits, axis=-1)
    eids = jax.lax.broadcasted_iota(jnp.int32, (x.shape[0], E), 1)
    onehot = (values[:, None] == eids).astype(jnp.float32)
    w_eff = jnp.dot(onehot, w_ref[...], preferred_element_type=jnp.float32)
    b_eff = jnp.dot(onehot, b_ref[...], preferred_element_type=jnp.float32)
    o_ref[...] = x * w_eff + b_eff


def _tc_kernel(inpt, W_router, w, b, skip=0):
    n, d = inpt.shape
    off = skip // BLOCK
    return pl.pallas_call(
        _tc_block_kernel,
        grid=((n - skip) // BLOCK,),
        in_specs=[
            pl.BlockSpec((BLOCK, d), lambda i: (i + off, 0)),
            pl.BlockSpec((d, E), lambda i: (0, 0)),
            pl.BlockSpec((E, d), lambda i: (0, 0)),
            pl.BlockSpec((E, d), lambda i: (0, 0)),
        ],
        out_specs=pl.BlockSpec((BLOCK, d), lambda i: (i + off, 0)),
        out_shape=jax.ShapeDtypeStruct((n, d), jnp.float32),
    )(inpt, W_router, w, b)


def _sc_body(x_hbm, wrT_hbm, w_hbm, b_hbm, out_hbm,
             xbuf, obuf, wrT_v, w_v, b_v, *, rc):
    y = out_hbm.shape[0]
    rpw = y // NW
    nchunks = rpw // rc
    RC = rc
    wid = lax.axis_index("s") * NC + lax.axis_index("c")

    pltpu.sync_copy(wrT_hbm, wrT_v)
    pltpu.sync_copy(w_hbm, w_v)
    pltpu.sync_copy(b_hbm, b_v)

    def chunk_body(c, _):
        base = wid * rpw + c * RC
        pltpu.sync_copy(x_hbm.at[pl.ds(base, RC)], xbuf)

        def group_body(g, _):
            r0 = g * RG
            acc = [[jnp.zeros((16,), jnp.float32) for _ in range(E)]
                   for _ in range(RG)]
            for j in range(NSL):
                sl = pl.ds(j * 16, 16)
                wr = [wrT_v[e, sl] for e in range(E)]
                for q in range(RG):
                    u = lax.bitcast_convert_type(xbuf[r0 + q, sl], jnp.uint32)
                    u = (u + jnp.uint32(0x7FFF) + ((u >> 16) & jnp.uint32(1)))
                    u = u & jnp.uint32(0xFFFF0000)
                    xv = lax.bitcast_convert_type(u, jnp.float32)
                    for e in range(E):
                        acc[q][e] = acc[q][e] + xv * wr[e]
            for q in range(RG):
                s = [jnp.sum(acc[q][e]) for e in range(E)]
                best = s[0]
                beste = jnp.int32(0)
                for e in range(1, E):
                    upd = s[e] > best
                    best = jnp.where(upd, s[e], best)
                    beste = jnp.where(upd, jnp.int32(e), beste)
                for j in range(NSL):
                    sl = pl.ds(j * 16, 16)
                    obuf[r0 + q, sl] = (xbuf[r0 + q, sl] * w_v[beste, sl]
                                        + b_v[beste, sl])
            return 0

        lax.fori_loop(0, RC // RG, group_body, 0)
        pltpu.sync_copy(obuf, out_hbm.at[pl.ds(base, RC)])
        return 0

    lax.fori_loop(0, nchunks, chunk_body, 0)


def _sc_kernel(inpt, W_router, w, b, y=None):
    n, d = inpt.shape
    if y is None:
        y = n
    rc = min(RC, y // NW)
    wrT = W_router.T.reshape(E, d)
    u = lax.bitcast_convert_type(wrT, jnp.uint32)
    u = (u + jnp.uint32(0x7FFF) + ((u >> 16) & jnp.uint32(1))) & jnp.uint32(0xFFFF0000)
    wrT = lax.bitcast_convert_type(u, jnp.float32)
    mesh = plsc.VectorSubcoreMesh(core_axis_name="c", subcore_axis_name="s",
                                  num_cores=NC, num_subcores=NS)
    run = pl.kernel(
        functools.partial(_sc_body, rc=rc),
        out_type=jax.ShapeDtypeStruct((y, d), jnp.float32),
        mesh=mesh,
        scratch_types=[
            pltpu.VMEM((rc, d), jnp.float32),
            pltpu.VMEM((rc, d), jnp.float32),
            pltpu.VMEM((E, d), jnp.float32),
            pltpu.VMEM((E, d), jnp.float32),
            pltpu.VMEM((E, d), jnp.float32),
        ],
        compiler_params=pltpu.CompilerParams(needs_layout_passes=False),
    )
    return run(inpt, wrT, w, b)


SC_ROWS = 1024


def kernel(inpt, W_router, w, b):
    tc_full = _tc_kernel(inpt, W_router, w, b, skip=SC_ROWS)
    sc_part = _sc_kernel(inpt, W_router, w, b, y=SC_ROWS)
    return lax.dynamic_update_slice(tc_full, sc_part, (0, 0))
